# baseline (speedup 1.0000x reference)
import functools

import jax
import jax.numpy as jnp
from jax import lax
from jax.experimental import pallas as pl
from jax.experimental.pallas import tpu as pltpu
from jax.experimental.pallas import tpu_sc as plsc

N_USER = 1000000
N_ITEM = 100000
N_INP = 64
BATCH = 16384

_info = plsc.get_sparse_core_info()
_NC, _NS = _info.num_cores, _info.num_subcores
_NW = _NC * _NS
_BPW = BATCH // _NW
_C = 256
_CI = 128


def _gather_body(user3, item3, nid_u_hbm, nid_i_hbm,
                 out_u_hbm, out_i_hbm,
                 idx_u, idx_i,
                 rows_u0, rows_u1, rows_i0, rows_i1,
                 sem_u0, sem_u1, sem_i0, sem_i1):
    wid = lax.axis_index("s") * _NC + lax.axis_index("c")
    base = wid * _BPW
    pltpu.sync_copy(nid_u_hbm.at[pl.ds(base, _BPW)], idx_u)
    pltpu.sync_copy(nid_i_hbm.at[pl.ds(base, _BPW)], idx_i)

    def burst(tab, idx, rows, sem, off, cnt):
        def issue(k, c):
            vec = idx[pl.ds(off + k * 16, 16)]
            for i in range(16):
                v = vec[i]
                pltpu.make_async_copy(tab.at[v],
                                      rows.at[k * 16 + i], sem).start()
            return c
        lax.fori_loop(0, cnt // 16, issue, 0)

    def drain_write(rows, sem, out, off, cnt):
        pltpu.make_async_copy(out.at[pl.ds(base + off, cnt)], rows, sem).wait()
        pltpu.sync_copy(rows, out.at[pl.ds(base + off, cnt)])

    burst(user3, idx_u, rows_u0, sem_u0, 0, _C)
    burst(user3, idx_u, rows_u1, sem_u1, _C, _C)
    burst(item3, idx_i, rows_i0, sem_i0, 0, _CI)
    burst(item3, idx_i, rows_i1, sem_i1, _CI, _CI)
    drain_write(rows_i0, sem_i0, out_i_hbm, 0, _CI)
    burst(item3, idx_i, rows_i0, sem_i0, 2 * _CI, _CI)
    drain_write(rows_i1, sem_i1, out_i_hbm, _CI, _CI)
    burst(item3, idx_i, rows_i1, sem_i1, 3 * _CI, _CI)
    drain_write(rows_u0, sem_u0, out_u_hbm, 0, _C)
    drain_write(rows_u1, sem_u1, out_u_hbm, _C, _C)
    drain_write(rows_i0, sem_i0, out_i_hbm, 2 * _CI, _CI)
    drain_write(rows_i1, sem_i1, out_i_hbm, 3 * _CI, _CI)


@jax.jit
def kernel(embed_user, embed_item, nid_user, nid_item):
    mesh = plsc.VectorSubcoreMesh(core_axis_name="c", subcore_axis_name="s")
    run = functools.partial(
        pl.kernel,
        mesh=mesh,
        out_type=(
            jax.ShapeDtypeStruct((BATCH, N_INP), jnp.float32),
            jax.ShapeDtypeStruct((BATCH, N_INP), jnp.float32),
        ),
        scratch_types=[
            pltpu.VMEM((_BPW,), jnp.int32),
            pltpu.VMEM((_BPW,), jnp.int32),
            pltpu.VMEM((_C, N_INP), jnp.float32),
            pltpu.VMEM((_C, N_INP), jnp.float32),
            pltpu.VMEM((_CI, N_INP), jnp.float32),
            pltpu.VMEM((_CI, N_INP), jnp.float32),
            pltpu.SemaphoreType.DMA,
            pltpu.SemaphoreType.DMA,
            pltpu.SemaphoreType.DMA,
            pltpu.SemaphoreType.DMA,
        ],
    )(_gather_body)
    return run(embed_user, embed_item, nid_user, nid_item)


# speedup vs baseline: 1.0024x; 1.0024x over previous
import functools

import jax
import jax.numpy as jnp
from jax import lax
from jax.experimental import pallas as pl
from jax.experimental.pallas import tpu as pltpu
from jax.experimental.pallas import tpu_sc as plsc

N_USER = 1000000
N_ITEM = 100000
N_INP = 64
BATCH = 16384

_info = plsc.get_sparse_core_info()
_NC, _NS = _info.num_cores, _info.num_subcores
_NW = _NC * _NS
_BPW = BATCH // _NW
_C = 256
_CI = 128


def _gather_body(user3, item3, nid_u_hbm, nid_i_hbm,
                 out_u_hbm, out_i_hbm,
                 idx_u, idx_i,
                 rows_u0, rows_u1, rows_i0, rows_i1,
                 sem_u0, sem_u1, sem_i0, sem_i1):
    wid = lax.axis_index("s") * _NC + lax.axis_index("c")
    base = wid * _BPW
    pltpu.sync_copy(nid_u_hbm.at[pl.ds(base, _BPW)], idx_u)
    pltpu.sync_copy(nid_i_hbm.at[pl.ds(base, _BPW)], idx_i)

    def burst(tab, idx, rows, sem, off, cnt):
        def issue(k, c):
            vec = idx[pl.ds(off + k * 16, 16)]
            for i in range(16):
                v = vec[i]
                pltpu.make_async_copy(tab.at[v],
                                      rows.at[k * 16 + i], sem).start()
            return c
        lax.fori_loop(0, cnt // 16, issue, 0)

    def drain_write(rows, sem, out, off, cnt):
        pltpu.make_async_copy(out.at[pl.ds(base + off, cnt)], rows, sem).wait()
        pltpu.sync_copy(rows, out.at[pl.ds(base + off, cnt)])

    burst(user3, idx_u, rows_u0, sem_u0, 0, _C)
    burst(user3, idx_u, rows_u1, sem_u1, _C, _C)
    burst(item3, idx_i, rows_i0, sem_i0, 0, _CI)
    burst(item3, idx_i, rows_i1, sem_i1, _CI, _CI)
    drain_write(rows_i0, sem_i0, out_i_hbm, 0, _CI)
    burst(item3, idx_i, rows_i0, sem_i0, 2 * _CI, _CI)
    drain_write(rows_i1, sem_i1, out_i_hbm, _CI, _CI)
    burst(item3, idx_i, rows_i1, sem_i1, 3 * _CI, _CI)
    drain_write(rows_u0, sem_u0, out_u_hbm, 0, _C)
    drain_write(rows_u1, sem_u1, out_u_hbm, _C, _C)
    drain_write(rows_i0, sem_i0, out_i_hbm, 2 * _CI, _CI)
    drain_write(rows_i1, sem_i1, out_i_hbm, 3 * _CI, _CI)


@jax.jit
def kernel(embed_user, embed_item, nid_user, nid_item):
    mesh = plsc.VectorSubcoreMesh(core_axis_name="c", subcore_axis_name="s")
    run = functools.partial(
        pl.kernel,
        mesh=mesh,
        out_type=(
            jax.ShapeDtypeStruct((BATCH, N_INP), jnp.float32),
            jax.ShapeDtypeStruct((BATCH, N_INP), jnp.float32),
        ),
        scratch_types=[
            pltpu.VMEM((_BPW,), jnp.int32),
            pltpu.VMEM((_BPW,), jnp.int32),
            pltpu.VMEM((_C, N_INP), jnp.float32),
            pltpu.VMEM((_C, N_INP), jnp.float32),
            pltpu.VMEM((_CI, N_INP), jnp.float32),
            pltpu.VMEM((_CI, N_INP), jnp.float32),
            pltpu.SemaphoreType.DMA,
            pltpu.SemaphoreType.DMA,
            pltpu.SemaphoreType.DMA,
            pltpu.SemaphoreType.DMA,
        ],
        compiler_params=pltpu.CompilerParams(use_tc_tiling_on_sc=True),
    )(_gather_body)
    return run(embed_user, embed_item, nid_user, nid_item)
